# Initial kernel scaffold; baseline (speedup 1.0000x reference)
import jax
import jax.numpy as jnp
from jax.experimental import pallas as pl


def kernel(subtree_batch, subtree_lens_batch, emb_ind_batch, emb_table, W_tree, b_tree, W_cls, b_cls):
    raise NotImplementedError("write your pallas kernel here")


# speedup vs baseline: 4.2743x; 4.2743x over previous
import functools

import jax
import jax.numpy as jnp
from jax import lax
from jax.experimental import pallas as pl
from jax.experimental.pallas import tpu as pltpu
from jax.experimental.pallas import tpu_sc as plsc

_B = 1024
_N = 200
_E = 64
_L = 16
_NTILES = 32
_EPT = _B // _NTILES
_G = 8
_NG = _EPT // _G


def _splat_i(x):
    return jnp.full((_L,), x, dtype=jnp.int32)


def _tree_kernel(subtree_hbm, embind_hbm, slens_hbm, emb_hbm, wlr_hbm,
                 btree_hbm, wcls_hbm, bcls_hbm, out_hbm,
                 reps_v, subtree_v, embind_v, slens_v, wlr_v, btree_v,
                 wcls_v, bcls_v, out_stage, sem):
    nc = 2
    wid = lax.axis_index("s") * nc + lax.axis_index("c")

    pltpu.sync_copy(wlr_hbm, wlr_v)
    pltpu.sync_copy(btree_hbm, btree_v)
    pltpu.sync_copy(wcls_hbm, wcls_v)
    pltpu.sync_copy(bcls_hbm, bcls_v)

    iota = lax.iota(jnp.int32, _L)

    def group_body(g, carry):
        e0 = wid * _EPT + g * _G
        grp = wid * _NG + g

        pltpu.sync_copy(subtree_hbm.at[pl.ds(e0 * (_N * 3), _G * _N * 3)],
                        subtree_v)
        pltpu.sync_copy(embind_hbm.at[grp], embind_v)
        pltpu.sync_copy(slens_hbm.at[grp], slens_v)

        for j in range(16):
            pltpu.async_copy(emb_hbm.at[embind_v.at[j]],
                             reps_v.at[pl.ds(j * 100, 100)], sem).wait()

        def example_body(b, carry2):
            row0 = b * _N
            sbase = b * (_N * 3)
            slen_vec = plsc.load_gather(slens_v, [_splat_i(b)])

            def step_body(i, carry3):
                ibase = _splat_i(sbase) + _splat_i(i) * 3
                parent = plsc.load_gather(subtree_v, [ibase])
                left = plsc.load_gather(subtree_v, [ibase + 1])
                right = plsc.load_gather(subtree_v, [ibase + 2])

                row_l = _splat_i(row0) + left
                row_r = _splat_i(row0) + right

                acc = [btree_v[pl.ds(c * _L, _L)] for c in range(4)]
                for k in range(_E):
                    xb = plsc.load_gather(reps_v, [row_l, _splat_i(k)])
                    for c in range(4):
                        acc[c] = acc[c] + wlr_v[k, pl.ds(c * _L, _L)] * xb
                for k in range(_E):
                    xb = plsc.load_gather(reps_v, [row_r, _splat_i(k)])
                    for c in range(4):
                        acc[c] = acc[c] + wlr_v[_E + k, pl.ds(c * _L, _L)] * xb

                cond = jnp.logical_and(left != right, _splat_i(i) < slen_vec)
                row_p = _splat_i(row0) + parent
                for c in range(4):
                    y = jnp.maximum(acc[c], 0.0)
                    plsc.store_scatter(reps_v, [row_p, iota + c * _L], y,
                                       mask=cond)
                return carry3

            lax.fori_loop(0, _N, step_body, 0)

            acc_o = bcls_v[...]
            for k in range(_E):
                xb = plsc.load_gather(reps_v, [_splat_i(row0), _splat_i(k)])
                acc_o = acc_o + wcls_v[k, pl.ds(0, _L)] * xb
            plsc.store_scatter(out_stage, [_splat_i(g * _G + b), iota], acc_o)
            return carry2

        lax.fori_loop(0, _G, example_body, 0)
        return carry

    lax.fori_loop(0, _NG, group_body, 0)
    pltpu.sync_copy(out_stage, out_hbm.at[pl.ds(wid * _EPT, _EPT)])


@jax.jit
def _run(subtree_flat, embind_g, slens_pad, emb_table, w_lr, b_tree,
         wcls_pad, bcls_pad):
    mesh = plsc.VectorSubcoreMesh(core_axis_name="c", subcore_axis_name="s")
    f = functools.partial(
        pl.kernel,
        mesh=mesh,
        compiler_params=pltpu.CompilerParams(needs_layout_passes=False,
                                             use_tc_tiling_on_sc=False),
        out_type=jax.ShapeDtypeStruct((_B, _L), jnp.float32),
        scratch_types=[
            pltpu.VMEM((_G * _N, _E), jnp.float32),
            pltpu.VMEM((_G * _N * 3,), jnp.int32),
            pltpu.VMEM((16, 100), jnp.int32),
            pltpu.VMEM((_L,), jnp.int32),
            pltpu.VMEM((2 * _E, _E), jnp.float32),
            pltpu.VMEM((_E,), jnp.float32),
            pltpu.VMEM((_E, _L), jnp.float32),
            pltpu.VMEM((_L,), jnp.float32),
            pltpu.VMEM((_EPT, _L), jnp.float32),
            pltpu.SemaphoreType.DMA,
        ],
    )(_tree_kernel)
    return f(subtree_flat, embind_g, slens_pad, emb_table, w_lr, b_tree,
             wcls_pad, bcls_pad)


def kernel(subtree_batch, subtree_lens_batch, emb_ind_batch, emb_table,
           W_tree, b_tree, W_cls, b_cls):
    subtree_flat = subtree_batch.astype(jnp.int32).reshape(-1)
    embind_g = emb_ind_batch.astype(jnp.int32).reshape(_B // _G, 16, 100)
    slens = subtree_lens_batch.astype(jnp.int32).reshape(_B // _G, _G)
    slens_pad = jnp.pad(slens, ((0, 0), (0, _L - _G)))
    w_lr = W_tree.T
    wcls_pad = jnp.pad(W_cls, ((0, _L - 5), (0, 0))).T
    bcls_pad = jnp.pad(b_cls, (0, _L - 5))
    out = _run(subtree_flat, embind_g, slens_pad, emb_table, w_lr,
               b_tree, wcls_pad, bcls_pad)
    return out[:, :5]
